# baseline (speedup 1.0000x reference)
import jax
import jax.numpy as jnp
from jax import lax
from jax.experimental import pallas as pl
from jax.experimental.pallas import tpu as pltpu
from jax.experimental.pallas import tpu_sc as plsc

BATCH = 16384
DIM = 32
VOCAB = 1000000
NUM_WORKERS = 32
ROWS_PER_WORKER = BATCH // NUM_WORKERS
HALF = ROWS_PER_WORKER // 2
LANES = 16
CHUNK_ITEMS = LANES
CHUNKS_PER_HALF = HALF // CHUNK_ITEMS
ELEMS_PER_CHUNK = CHUNK_ITEMS * DIM
DMA_ELEMS = 128
DMAS_PER_CHUNK = ELEMS_PER_CHUNK // DMA_ELEMS
NSEM = 3


def _body(uidx_hbm, pidx_hbm, nidx_hbm, utab_hbm, itab_hbm, out_hbm,
          idx2d_v, uidx_v, pidx_v, nidx_v,
          uaddr_v, paddr_v, naddr_v, udata_v, pdata_v, ndata_v,
          outbuf_v, *sems):
    cid = lax.axis_index("c")
    sid = lax.axis_index("s")
    wid = sid * 2 + cid
    base = wid * ROWS_PER_WORKER

    col0 = jnp.zeros((LANES,), jnp.int32)
    col1 = jnp.ones((LANES,), jnp.int32)
    lane_iota = lax.iota(jnp.int32, LANES)

    row_sl = pl.ds(base, ROWS_PER_WORKER)
    for src, flat in ((uidx_hbm, uidx_v), (pidx_hbm, pidx_v), (nidx_hbm, nidx_v)):
        pltpu.sync_copy(src.at[row_sl], idx2d_v)
        for c in range(ROWS_PER_WORKER // LANES):
            vals = plsc.load_gather(idx2d_v, [c * LANES + lane_iota, col0])
            flat[pl.ds(c * LANES, LANES)] = vals

    tables = ((uidx_v, uaddr_v, udata_v, utab_hbm),
              (pidx_v, paddr_v, pdata_v, itab_hbm),
              (nidx_v, naddr_v, ndata_v, itab_hbm))

    def build_addresses(half):
        for flat, addr, _, _ in tables:
            for c in range(CHUNKS_PER_HALF):
                v16 = flat[pl.ds(half * HALF + c * LANES, LANES)]
                for d in range(DIM):
                    addr[pl.ds((c * DIM + d) * LANES, LANES)] = v16 + d * VOCAB

    def fire_chunk(c):
        hs = []
        for _, addr, data, tab in tables:
            for k in range(DMAS_PER_CHUNK):
                sl = pl.ds(c * ELEMS_PER_CHUNK + k * DMA_ELEMS, DMA_ELEMS)
                hs.append(pltpu.async_copy(
                    tab.at[addr.at[sl]], data.at[sl], sems[c % NSEM]))
        return hs

    def dot_chunk(half, c):
        ebase = c * ELEMS_PER_CHUNK

        def per_d(d, carry):
            accp, accn = carry
            sl = pl.ds(ebase + d * LANES, LANES)
            u = udata_v[sl]
            p = pdata_v[sl]
            n = ndata_v[sl]
            return accp + u * p, accn + u * n

        zero = jnp.zeros((LANES,), jnp.float32)
        accp, accn = lax.fori_loop(0, DIM, per_d, (zero, zero))
        lanes = half * HALF + c * LANES + lane_iota
        plsc.store_scatter(outbuf_v, [lanes, col0], accp)
        plsc.store_scatter(outbuf_v, [lanes, col1], accn)

    for half in range(2):
        build_addresses(half)
        handles = {0: fire_chunk(0), 1: fire_chunk(1)}
        for c in range(CHUNKS_PER_HALF):
            for h in handles.pop(c):
                h.wait()
            if c + 2 < CHUNKS_PER_HALF:
                handles[c + 2] = fire_chunk(c + 2)
            dot_chunk(half, c)

    pltpu.sync_copy(outbuf_v, out_hbm.at[pl.ds(base, ROWS_PER_WORKER)])


def kernel(user_inputs, pos_inputs, neg_inputs, user_table, item_table):
    mesh = plsc.VectorSubcoreMesh(core_axis_name="c", subcore_axis_name="s")
    nelem_half = CHUNKS_PER_HALF * ELEMS_PER_CHUNK
    run = pl.kernel(
        _body,
        out_type=jax.ShapeDtypeStruct((BATCH, 2), jnp.float32),
        mesh=mesh,
        scratch_types=[
            pltpu.VMEM((ROWS_PER_WORKER, 1), jnp.int32),
            pltpu.VMEM((ROWS_PER_WORKER,), jnp.int32),
            pltpu.VMEM((ROWS_PER_WORKER,), jnp.int32),
            pltpu.VMEM((ROWS_PER_WORKER,), jnp.int32),
            pltpu.VMEM((nelem_half,), jnp.int32),
            pltpu.VMEM((nelem_half,), jnp.int32),
            pltpu.VMEM((nelem_half,), jnp.int32),
            pltpu.VMEM((nelem_half,), jnp.float32),
            pltpu.VMEM((nelem_half,), jnp.float32),
            pltpu.VMEM((nelem_half,), jnp.float32),
            pltpu.VMEM((ROWS_PER_WORKER, 2), jnp.float32),
        ] + [pltpu.SemaphoreType.DMA] * NSEM,
        compiler_params=pltpu.CompilerParams(
            needs_layout_passes=False,
            use_tc_tiling_on_sc=False,
        ),
    )
    uflat = user_table.T.reshape(VOCAB * DIM)
    iflat = item_table.T.reshape(VOCAB * DIM)
    return run(user_inputs, pos_inputs, neg_inputs, uflat, iflat)


# speedup vs baseline: 3.4628x; 3.4628x over previous
import jax
import jax.numpy as jnp
from jax import lax
from jax.experimental import pallas as pl
from jax.experimental.pallas import tpu as pltpu
from jax.experimental.pallas import tpu_sc as plsc

BATCH = 16384
DIM = 32
VOCAB = 1000000
NUM_WORKERS = 32
ROWS_PER_WORKER = BATCH // NUM_WORKERS
HALF = ROWS_PER_WORKER // 2
LANES = 16
CHUNKS_PER_HALF = HALF // LANES
DMA_ELEMS = 128
DMAS_PER_D = HALF // DMA_ELEMS
NSEM = 3


def _body(uidx_hbm, pidx_hbm, nidx_hbm, *rest):
    utabs = rest[:DIM]
    itabs = rest[DIM:2 * DIM]
    out_hbm = rest[2 * DIM]
    (idx2d_v, uidx_v, pidx_v, nidx_v,
     udata_v, pdata_v, ndata_v, outbuf_v) = rest[2 * DIM + 1:2 * DIM + 9]
    sems = rest[2 * DIM + 9:]

    cid = lax.axis_index("c")
    sid = lax.axis_index("s")
    wid = sid * 2 + cid
    base = wid * ROWS_PER_WORKER

    col0 = jnp.zeros((LANES,), jnp.int32)
    col1 = jnp.ones((LANES,), jnp.int32)
    lane_iota = lax.iota(jnp.int32, LANES)

    row_sl = pl.ds(base, ROWS_PER_WORKER)
    for src, flat in ((uidx_hbm, uidx_v), (pidx_hbm, pidx_v), (nidx_hbm, nidx_v)):
        pltpu.sync_copy(src.at[row_sl], idx2d_v)
        for c in range(ROWS_PER_WORKER // LANES):
            vals = plsc.load_gather(idx2d_v, [c * LANES + lane_iota, col0])
            flat[pl.ds(c * LANES, LANES)] = vals

    tables = ((uidx_v, udata_v, utabs),
              (pidx_v, pdata_v, itabs),
              (nidx_v, ndata_v, itabs))

    def fire_d(half, d):
        hs = []
        for flat, data, tabs in tables:
            for k in range(DMAS_PER_D):
                isl = pl.ds(half * HALF + k * DMA_ELEMS, DMA_ELEMS)
                dsl = pl.ds(d * HALF + k * DMA_ELEMS, DMA_ELEMS)
                hs.append(pltpu.async_copy(
                    tabs[d].at[flat.at[isl]], data.at[dsl], sems[d % NSEM]))
        return hs

    zero = jnp.zeros((LANES,), jnp.float32)
    for half in range(2):
        handles = {0: fire_d(half, 0), 1: fire_d(half, 1)}
        accp = [zero] * CHUNKS_PER_HALF
        accn = [zero] * CHUNKS_PER_HALF
        for d in range(DIM):
            for h in handles.pop(d):
                h.wait()
            if d + 2 < DIM:
                handles[d + 2] = fire_d(half, d + 2)
            for c in range(CHUNKS_PER_HALF):
                sl = pl.ds(d * HALF + c * LANES, LANES)
                u = udata_v[sl]
                p = pdata_v[sl]
                n = ndata_v[sl]
                accp[c] = accp[c] + u * p
                accn[c] = accn[c] + u * n
        for c in range(CHUNKS_PER_HALF):
            lanes = half * HALF + c * LANES + lane_iota
            plsc.store_scatter(outbuf_v, [lanes, col0], accp[c])
            plsc.store_scatter(outbuf_v, [lanes, col1], accn[c])

    pltpu.sync_copy(outbuf_v, out_hbm.at[pl.ds(base, ROWS_PER_WORKER)])


def kernel(user_inputs, pos_inputs, neg_inputs, user_table, item_table):
    mesh = plsc.VectorSubcoreMesh(core_axis_name="c", subcore_axis_name="s")
    nelem_half = DIM * HALF
    run = pl.kernel(
        _body,
        out_type=jax.ShapeDtypeStruct((BATCH, 2), jnp.float32),
        mesh=mesh,
        scratch_types=[
            pltpu.VMEM((ROWS_PER_WORKER, 1), jnp.int32),
            pltpu.VMEM((ROWS_PER_WORKER,), jnp.int32),
            pltpu.VMEM((ROWS_PER_WORKER,), jnp.int32),
            pltpu.VMEM((ROWS_PER_WORKER,), jnp.int32),
            pltpu.VMEM((nelem_half,), jnp.float32),
            pltpu.VMEM((nelem_half,), jnp.float32),
            pltpu.VMEM((nelem_half,), jnp.float32),
            pltpu.VMEM((ROWS_PER_WORKER, 2), jnp.float32),
        ] + [pltpu.SemaphoreType.DMA] * NSEM,
        compiler_params=pltpu.CompilerParams(
            needs_layout_passes=False,
            use_tc_tiling_on_sc=False,
        ),
    )
    ut = user_table.T
    it = item_table.T
    uplanes = [lax.slice_in_dim(ut, d, d + 1, axis=0).reshape(VOCAB)
               for d in range(DIM)]
    iplanes = [lax.slice_in_dim(it, d, d + 1, axis=0).reshape(VOCAB)
               for d in range(DIM)]
    return run(user_inputs, pos_inputs, neg_inputs, *uplanes, *iplanes)


# speedup vs baseline: 5.4471x; 1.5730x over previous
import jax
import jax.numpy as jnp
from jax import lax
from jax.experimental import pallas as pl
from jax.experimental.pallas import tpu as pltpu
from jax.experimental.pallas import tpu_sc as plsc

BATCH = 16384
DIM = 32
VOCAB = 1000000
GROUP = 128 // DIM
NUM_WORKERS = 32
ROWS_PER_WORKER = BATCH // NUM_WORKERS
HALF = ROWS_PER_WORKER // 2
LANES = 16
CHUNKS_PER_HALF = HALF // LANES
DMA_ROWS = 128
DMAS_PER_HALF = HALF // DMA_ROWS


def _body(uidx_hbm, pidx_hbm, nidx_hbm, utab_hbm, itab_hbm, out_hbm,
          idx2d_v, uidx_v, pidx_v, nidx_v, ugrp_v, pgrp_v, ngrp_v,
          urows_v, prows_v, nrows_v, outbuf_v, *sems):
    cid = lax.axis_index("c")
    sid = lax.axis_index("s")
    wid = sid * 2 + cid
    base = wid * ROWS_PER_WORKER

    col0 = jnp.zeros((LANES,), jnp.int32)
    col1 = jnp.ones((LANES,), jnp.int32)
    lane_iota = lax.iota(jnp.int32, LANES)

    row_sl = pl.ds(base, ROWS_PER_WORKER)
    for src, flat, grp in ((uidx_hbm, uidx_v, ugrp_v),
                           (pidx_hbm, pidx_v, pgrp_v),
                           (nidx_hbm, nidx_v, ngrp_v)):
        pltpu.sync_copy(src.at[row_sl], idx2d_v)
        for c in range(ROWS_PER_WORKER // LANES):
            vals = plsc.load_gather(idx2d_v, [c * LANES + lane_iota, col0])
            flat[pl.ds(c * LANES, LANES)] = vals
            grp[pl.ds(c * LANES, LANES)] = lax.shift_right_logical(vals, 2)

    tables = ((uidx_v, ugrp_v, urows_v, utab_hbm),
              (pidx_v, pgrp_v, prows_v, itab_hbm),
              (nidx_v, ngrp_v, nrows_v, itab_hbm))

    def fire_half(half):
        hs = []
        for _, grp, rows, tab in tables:
            for k in range(DMAS_PER_HALF):
                isl = pl.ds(half * HALF + k * DMA_ROWS, DMA_ROWS)
                dsl = pl.ds(k * DMA_ROWS, DMA_ROWS)
                hs.append(pltpu.async_copy(
                    tab.at[grp.at[isl]], rows.at[dsl], sems[half]))
        return hs

    def dot_half(half):
        for c in range(CHUNKS_PER_HALF):
            slots = c * LANES + lane_iota
            accp = jnp.zeros((LANES,), jnp.float32)
            accn = jnp.zeros((LANES,), jnp.float32)
            offs = []
            for flat, _, _, _ in tables:
                raw = flat[pl.ds(half * HALF + c * LANES, LANES)]
                offs.append((raw & (GROUP - 1)) * DIM)
            uoff, poff, noff = offs
            for d in range(DIM):
                u = plsc.load_gather(urows_v, [slots, uoff + d])
                p = plsc.load_gather(prows_v, [slots, poff + d])
                n = plsc.load_gather(nrows_v, [slots, noff + d])
                accp = accp + u * p
                accn = accn + u * n
            lanes = half * HALF + c * LANES + lane_iota
            plsc.store_scatter(outbuf_v, [lanes, col0], accp)
            plsc.store_scatter(outbuf_v, [lanes, col1], accn)

    h0 = fire_half(0)
    h1 = fire_half(1)
    for h in h0:
        h.wait()
    dot_half(0)
    for h in h1:
        h.wait()
    dot_half(1)

    pltpu.sync_copy(outbuf_v, out_hbm.at[pl.ds(base, ROWS_PER_WORKER)])


def kernel(user_inputs, pos_inputs, neg_inputs, user_table, item_table):
    mesh = plsc.VectorSubcoreMesh(core_axis_name="c", subcore_axis_name="s")
    run = pl.kernel(
        _body,
        out_type=jax.ShapeDtypeStruct((BATCH, 2), jnp.float32),
        mesh=mesh,
        scratch_types=[
            pltpu.VMEM((ROWS_PER_WORKER, 1), jnp.int32),
            pltpu.VMEM((ROWS_PER_WORKER,), jnp.int32),
            pltpu.VMEM((ROWS_PER_WORKER,), jnp.int32),
            pltpu.VMEM((ROWS_PER_WORKER,), jnp.int32),
            pltpu.VMEM((ROWS_PER_WORKER,), jnp.int32),
            pltpu.VMEM((ROWS_PER_WORKER,), jnp.int32),
            pltpu.VMEM((ROWS_PER_WORKER,), jnp.int32),
            pltpu.VMEM((HALF, 128), jnp.float32),
            pltpu.VMEM((HALF, 128), jnp.float32),
            pltpu.VMEM((HALF, 128), jnp.float32),
            pltpu.VMEM((ROWS_PER_WORKER, 2), jnp.float32),
        ] + [pltpu.SemaphoreType.DMA] * 2,
        compiler_params=pltpu.CompilerParams(
            needs_layout_passes=False,
            use_tc_tiling_on_sc=False,
        ),
    )
    ugrp = user_table.reshape(VOCAB // GROUP, 128)
    igrp = item_table.reshape(VOCAB // GROUP, 128)
    return run(user_inputs, pos_inputs, neg_inputs, ugrp, igrp)
